# baseline (speedup 1.0000x reference)
import functools

import jax
import jax.numpy as jnp
from jax import lax
from jax.experimental import pallas as pl
from jax.experimental.pallas import tpu as pltpu
from jax.experimental.pallas import tpu_sc as plsc

INP_DIM = 1024
EMB_DIM = 64
N_PROJ = 16
NUM_BINS = 20
NCOLS = (NUM_BINS + 1) * N_PROJ

TOK_BLOCK = 2048
NCHUNK = 4


def _consts():
    resolution = 2.0 / NUM_BINS
    grid = jnp.linspace(-1.0, 1.0, NUM_BINS + 1)[:-1] + 0.5 * resolution
    lower = jnp.concatenate([jnp.full((1,), -3.0, jnp.float32), grid])
    lower = jnp.tile(lower, (N_PROJ,)).reshape(1, NCOLS)
    p_of_col = jnp.arange(NCOLS, dtype=jnp.int32) // (NUM_BINS + 1)
    expand = (p_of_col[None, :] == jnp.arange(N_PROJ, dtype=jnp.int32)[:, None])
    expand = expand.astype(jnp.bfloat16)
    cp = ((jnp.arange(N_PROJ, dtype=jnp.float32) * (NUM_BINS + 1) - 1.0)
          * (EMB_DIM // 2)).reshape(1, N_PROJ)
    return lower, expand, cp


def _addr_body(x_ref, pm_ref, low_ref, ex_ref, seg_ref, cp_ref, addr_ref):
    xb = x_ref[...]
    ssq = jnp.sum(xb * xb, axis=1, keepdims=True)
    norm = jnp.maximum(jnp.sqrt(ssq), 1e-12)
    xn = xb / norm
    z = jnp.dot(xn.astype(jnp.bfloat16), pm_ref[...].astype(jnp.bfloat16),
                preferred_element_type=jnp.float32)
    ex = ex_ref[...]
    z_hi = z.astype(jnp.bfloat16)
    r1 = z - z_hi.astype(jnp.float32)
    z_mid = r1.astype(jnp.bfloat16)
    z_lo = (r1 - z_mid.astype(jnp.float32)).astype(jnp.bfloat16)
    zc = (jnp.dot(z_hi, ex, preferred_element_type=jnp.float32)
          + jnp.dot(z_mid, ex, preferred_element_type=jnp.float32)
          + jnp.dot(z_lo, ex, preferred_element_type=jnp.float32))
    ge = (zc > low_ref[...]).astype(jnp.bfloat16)
    cnt = jnp.dot(ge, seg_ref[...], preferred_element_type=jnp.float32)
    addr_ref[...] = (cnt * float(EMB_DIM // 2) + cp_ref[...]).astype(jnp.int32)


def _tc_addr(xf, projection_mat, chunk, nchunk):
    ntok = xf.shape[0]
    ctok = ntok // nchunk
    nblk = ctok // TOK_BLOCK
    lower, expand, cp = _consts()
    return pl.pallas_call(
        _addr_body,
        grid=(nblk,),
        in_specs=[
            pl.BlockSpec((TOK_BLOCK, INP_DIM),
                         lambda i, c=chunk, n=nblk: (c * n + i, 0)),
            pl.BlockSpec((INP_DIM, N_PROJ), lambda i: (0, 0)),
            pl.BlockSpec((1, NCOLS), lambda i: (0, 0)),
            pl.BlockSpec((N_PROJ, NCOLS), lambda i: (0, 0)),
            pl.BlockSpec((NCOLS, N_PROJ), lambda i: (0, 0)),
            pl.BlockSpec((1, N_PROJ), lambda i: (0, 0)),
        ],
        out_specs=pl.BlockSpec((TOK_BLOCK, N_PROJ), lambda i: (i, 0)),
        out_shape=jax.ShapeDtypeStruct((ctok, N_PROJ), jnp.int32),
        compiler_params=pltpu.CompilerParams(
            dimension_semantics=("arbitrary",),
        ),
    )(xf, projection_mat, lower, expand, jnp.transpose(expand), cp)


def _pack_table(emb_weight):
    wr = emb_weight.reshape(NCOLS, 2, 2, 16)
    lo = jax.lax.bitcast_convert_type(
        wr[:, :, 0, :].astype(jnp.bfloat16), jnp.uint16).astype(jnp.uint32)
    hi = jax.lax.bitcast_convert_type(
        wr[:, :, 1, :].astype(jnp.bfloat16), jnp.uint16).astype(jnp.uint32)
    words = lo | (hi << 16)
    return jax.lax.bitcast_convert_type(words, jnp.int32).reshape(-1)


def _sc_bag(ptable, addr_flat, ntok):
    info = plsc.get_sparse_core_info()
    nc, ns = info.num_cores, info.num_subcores
    nw = nc * ns
    tpw = ntok // nw
    mesh = plsc.VectorSubcoreMesh(core_axis_name="c", subcore_axis_name="s")

    @functools.partial(
        pl.kernel,
        out_type=jax.ShapeDtypeStruct((ntok * EMB_DIM,), jnp.float32),
        mesh=mesh,
        scratch_types=[
            pltpu.VMEM((NCOLS * EMB_DIM // 2,), jnp.int32),
            pltpu.VMEM((tpw * N_PROJ,), jnp.int32),
            pltpu.VMEM((tpw * EMB_DIM,), jnp.float32),
        ],
        compiler_params=pltpu.CompilerParams(needs_layout_passes=False),
    )
    def bag(table_hbm, addr_hbm, out_hbm, table_v, addr_v, out_v):
        wid = lax.axis_index("s") * nc + lax.axis_index("c")
        pltpu.sync_copy(table_hbm, table_v)
        pltpu.sync_copy(
            addr_hbm.at[pl.ds(wid * (tpw * N_PROJ), tpw * N_PROJ)], addr_v)
        lane = lax.iota(jnp.int32, 16)
        offs = [lane, lane + 16]

        def body(t, carry):
            accs = [jnp.zeros((16,), jnp.float32) for _ in range(EMB_DIM // 16)]
            av = addr_v[pl.ds(t * N_PROJ, N_PROJ)]
            for p in range(N_PROJ):
                a = av[p]
                for g in range(2):
                    w = plsc.load_gather(table_v, [a + offs[g]])
                    pair = plsc.bitcast(w, jnp.bfloat16)
                    lo, hi = plsc.unpack(pair, format=plsc.PackFormat.INTERLEAVED)
                    accs[2 * g] = accs[2 * g] + lo
                    accs[2 * g + 1] = accs[2 * g + 1] + hi
            for c in range(EMB_DIM // 16):
                out_v[pl.ds(t * EMB_DIM + 16 * c, 16)] = accs[c] * (1.0 / N_PROJ)
            return carry

        lax.fori_loop(0, tpw, body, 0)
        pltpu.sync_copy(out_v, out_hbm.at[pl.ds(wid * (tpw * EMB_DIM),
                                                tpw * EMB_DIM)])

    return bag(ptable, addr_flat)


@jax.jit
def kernel(x, projection_mat, emb_weight):
    bs, seq_len, _ = x.shape
    ntok = bs * seq_len
    ctok = ntok // NCHUNK
    xf = x.reshape(ntok, INP_DIM)
    ptable = _pack_table(emb_weight)
    outs = []
    for i in range(NCHUNK):
        addr = _tc_addr(xf, projection_mat, i, NCHUNK)
        outs.append(_sc_bag(ptable, addr.reshape(-1), ctok))
    out = jnp.concatenate(outs)
    return out.reshape(bs, seq_len, EMB_DIM)


# speedup vs baseline: 1.0198x; 1.0198x over previous
import functools

import jax
import jax.numpy as jnp
from jax import lax
from jax.experimental import pallas as pl
from jax.experimental.pallas import tpu as pltpu
from jax.experimental.pallas import tpu_sc as plsc

INP_DIM = 1024
EMB_DIM = 64
N_PROJ = 16
NUM_BINS = 20
NCOLS = (NUM_BINS + 1) * N_PROJ

TOK_BLOCK = 1024
NCHUNK = 2


def _consts():
    resolution = 2.0 / NUM_BINS
    grid = jnp.linspace(-1.0, 1.0, NUM_BINS + 1)[:-1] + 0.5 * resolution
    lower = jnp.concatenate([jnp.full((1,), -3.0, jnp.float32), grid])
    lower = jnp.tile(lower, (N_PROJ,)).reshape(1, NCOLS)
    p_of_col = jnp.arange(NCOLS, dtype=jnp.int32) // (NUM_BINS + 1)
    expand = (p_of_col[None, :] == jnp.arange(N_PROJ, dtype=jnp.int32)[:, None])
    expand = expand.astype(jnp.bfloat16)
    cp = ((jnp.arange(N_PROJ, dtype=jnp.float32) * (NUM_BINS + 1) - 1.0)
          * (EMB_DIM // 2)).reshape(1, N_PROJ)
    return lower, expand, cp


def _addr_body(x_ref, pm_ref, low_ref, ex_ref, seg_ref, cp_ref, addr_ref):
    xb = x_ref[...]
    ssq = jnp.sum(xb * xb, axis=1, keepdims=True)
    norm = jnp.maximum(jnp.sqrt(ssq), 1e-12)
    xn = xb / norm
    z = jnp.dot(xn.astype(jnp.bfloat16), pm_ref[...].astype(jnp.bfloat16),
                preferred_element_type=jnp.float32)
    ex = ex_ref[...]
    z_hi = z.astype(jnp.bfloat16)
    r1 = z - z_hi.astype(jnp.float32)
    z_mid = r1.astype(jnp.bfloat16)
    z_lo = (r1 - z_mid.astype(jnp.float32)).astype(jnp.bfloat16)
    zc = (jnp.dot(z_hi, ex, preferred_element_type=jnp.float32)
          + jnp.dot(z_mid, ex, preferred_element_type=jnp.float32)
          + jnp.dot(z_lo, ex, preferred_element_type=jnp.float32))
    ge = (zc > low_ref[...]).astype(jnp.bfloat16)
    cnt = jnp.dot(ge, seg_ref[...], preferred_element_type=jnp.float32)
    addr_ref[...] = (cnt * float(EMB_DIM // 2) + cp_ref[...]).astype(jnp.int32)


def _tc_addr(xf, projection_mat, chunk, nchunk):
    ntok = xf.shape[0]
    ctok = ntok // nchunk
    nblk = ctok // TOK_BLOCK
    lower, expand, cp = _consts()
    return pl.pallas_call(
        _addr_body,
        grid=(nblk,),
        in_specs=[
            pl.BlockSpec((TOK_BLOCK, INP_DIM),
                         lambda i, c=chunk, n=nblk: (c * n + i, 0)),
            pl.BlockSpec((INP_DIM, N_PROJ), lambda i: (0, 0)),
            pl.BlockSpec((1, NCOLS), lambda i: (0, 0)),
            pl.BlockSpec((N_PROJ, NCOLS), lambda i: (0, 0)),
            pl.BlockSpec((NCOLS, N_PROJ), lambda i: (0, 0)),
            pl.BlockSpec((1, N_PROJ), lambda i: (0, 0)),
        ],
        out_specs=pl.BlockSpec((TOK_BLOCK, N_PROJ), lambda i: (i, 0)),
        out_shape=jax.ShapeDtypeStruct((ctok, N_PROJ), jnp.int32),
        compiler_params=pltpu.CompilerParams(
            dimension_semantics=("arbitrary",),
        ),
    )(xf, projection_mat, lower, expand, jnp.transpose(expand), cp)


def _pack_table(emb_weight):
    wr = emb_weight.reshape(NCOLS, 2, 2, 16)
    lo = jax.lax.bitcast_convert_type(
        wr[:, :, 0, :].astype(jnp.bfloat16), jnp.uint16).astype(jnp.uint32)
    hi = jax.lax.bitcast_convert_type(
        wr[:, :, 1, :].astype(jnp.bfloat16), jnp.uint16).astype(jnp.uint32)
    words = lo | (hi << 16)
    return jax.lax.bitcast_convert_type(words, jnp.int32).reshape(-1)


def _sc_bag(ptable, addr_flat, ntok):
    info = plsc.get_sparse_core_info()
    nc, ns = info.num_cores, info.num_subcores
    nw = nc * ns
    tpw = ntok // nw
    mesh = plsc.VectorSubcoreMesh(core_axis_name="c", subcore_axis_name="s")

    @functools.partial(
        pl.kernel,
        out_type=jax.ShapeDtypeStruct((ntok * EMB_DIM,), jnp.float32),
        mesh=mesh,
        scratch_types=[
            pltpu.VMEM((NCOLS * EMB_DIM // 2,), jnp.int32),
            pltpu.VMEM((tpw * N_PROJ,), jnp.int32),
            pltpu.VMEM((tpw * EMB_DIM,), jnp.float32),
        ],
        compiler_params=pltpu.CompilerParams(needs_layout_passes=False),
    )
    def bag(table_hbm, addr_hbm, out_hbm, table_v, addr_v, out_v):
        wid = lax.axis_index("s") * nc + lax.axis_index("c")
        pltpu.sync_copy(table_hbm, table_v)
        pltpu.sync_copy(
            addr_hbm.at[pl.ds(wid * (tpw * N_PROJ), tpw * N_PROJ)], addr_v)
        lane = lax.iota(jnp.int32, 16)
        offs = [lane, lane + 16]

        def body(t, carry):
            accs = [jnp.zeros((16,), jnp.float32) for _ in range(EMB_DIM // 16)]
            av = addr_v[pl.ds(t * N_PROJ, N_PROJ)]
            for p in range(N_PROJ):
                a = av[p]
                for g in range(2):
                    w = plsc.load_gather(table_v, [a + offs[g]])
                    pair = plsc.bitcast(w, jnp.bfloat16)
                    lo, hi = plsc.unpack(pair, format=plsc.PackFormat.INTERLEAVED)
                    accs[2 * g] = accs[2 * g] + lo
                    accs[2 * g + 1] = accs[2 * g + 1] + hi
            for c in range(EMB_DIM // 16):
                out_v[pl.ds(t * EMB_DIM + 16 * c, 16)] = accs[c] * (1.0 / N_PROJ)
            return carry

        lax.fori_loop(0, tpw, body, 0)
        pltpu.sync_copy(out_v, out_hbm.at[pl.ds(wid * (tpw * EMB_DIM),
                                                tpw * EMB_DIM)])

    return bag(ptable, addr_flat)


@jax.jit
def kernel(x, projection_mat, emb_weight):
    bs, seq_len, _ = x.shape
    ntok = bs * seq_len
    ctok = ntok // NCHUNK
    xf = x.reshape(ntok, INP_DIM)
    ptable = _pack_table(emb_weight)
    outs = []
    for i in range(NCHUNK):
        addr = _tc_addr(xf, projection_mat, i, NCHUNK)
        outs.append(_sc_bag(ptable, addr.reshape(-1), ctok))
    out = jnp.concatenate(outs)
    return out.reshape(bs, seq_len, EMB_DIM)


# speedup vs baseline: 1.1429x; 1.1207x over previous
import functools

import jax
import jax.numpy as jnp
from jax import lax
from jax.experimental import pallas as pl
from jax.experimental.pallas import tpu as pltpu
from jax.experimental.pallas import tpu_sc as plsc

INP_DIM = 1024
EMB_DIM = 64
N_PROJ = 16
NUM_BINS = 20
NCOLS = (NUM_BINS + 1) * N_PROJ

TOK_BLOCK = 1024
NCHUNK = 4


def _consts():
    resolution = 2.0 / NUM_BINS
    grid = jnp.linspace(-1.0, 1.0, NUM_BINS + 1)[:-1] + 0.5 * resolution
    lower = jnp.concatenate([jnp.full((1,), -3.0, jnp.float32), grid])
    lower = jnp.tile(lower, (N_PROJ,)).reshape(1, NCOLS)
    p_of_col = jnp.arange(NCOLS, dtype=jnp.int32) // (NUM_BINS + 1)
    seg = (p_of_col[:, None] == jnp.arange(N_PROJ, dtype=jnp.int32)[None, :])
    seg = seg.astype(jnp.bfloat16)
    cp = ((jnp.arange(N_PROJ, dtype=jnp.float32) * (NUM_BINS + 1) - 1.0)
          * (EMB_DIM // 2)).reshape(1, N_PROJ)
    return lower, seg, cp


def _addr_body(x_ref, pmx_ref, low_ref, seg_ref, cp_ref, addr_ref):
    xb = x_ref[...]
    ssq = jnp.sum(xb * xb, axis=1, keepdims=True)
    norm = jnp.maximum(jnp.sqrt(ssq), 1e-12)
    xn = xb / norm
    zc = jnp.dot(xn.astype(jnp.bfloat16), pmx_ref[...],
                 preferred_element_type=jnp.float32)
    ge = (zc > low_ref[...]).astype(jnp.bfloat16)
    cnt = jnp.dot(ge, seg_ref[...], preferred_element_type=jnp.float32)
    addr_ref[...] = (cnt * float(EMB_DIM // 2) + cp_ref[...]).astype(jnp.int32)


def _tc_addr(xf, projection_mat, chunk, nchunk):
    ntok = xf.shape[0]
    ctok = ntok // nchunk
    nblk = ctok // TOK_BLOCK
    lower, seg, cp = _consts()
    pmx = jnp.repeat(projection_mat.astype(jnp.bfloat16), NUM_BINS + 1, axis=1)
    return pl.pallas_call(
        _addr_body,
        grid=(nblk,),
        in_specs=[
            pl.BlockSpec((TOK_BLOCK, INP_DIM),
                         lambda i, c=chunk, n=nblk: (c * n + i, 0)),
            pl.BlockSpec((INP_DIM, NCOLS), lambda i: (0, 0)),
            pl.BlockSpec((1, NCOLS), lambda i: (0, 0)),
            pl.BlockSpec((NCOLS, N_PROJ), lambda i: (0, 0)),
            pl.BlockSpec((1, N_PROJ), lambda i: (0, 0)),
        ],
        out_specs=pl.BlockSpec((TOK_BLOCK, N_PROJ), lambda i: (i, 0)),
        out_shape=jax.ShapeDtypeStruct((ctok, N_PROJ), jnp.int32),
        compiler_params=pltpu.CompilerParams(
            dimension_semantics=("arbitrary",),
        ),
    )(xf, pmx, lower, seg, cp)


def _pack_table(emb_weight):
    wr = emb_weight.reshape(NCOLS, 2, 2, 16)
    lo = jax.lax.bitcast_convert_type(
        wr[:, :, 0, :].astype(jnp.bfloat16), jnp.uint16).astype(jnp.uint32)
    hi = jax.lax.bitcast_convert_type(
        wr[:, :, 1, :].astype(jnp.bfloat16), jnp.uint16).astype(jnp.uint32)
    words = lo | (hi << 16)
    return jax.lax.bitcast_convert_type(words, jnp.int32).reshape(-1)


def _sc_bag(ptable, addr_flat, ntok):
    info = plsc.get_sparse_core_info()
    nc, ns = info.num_cores, info.num_subcores
    nw = nc * ns
    tpw = ntok // nw
    mesh = plsc.VectorSubcoreMesh(core_axis_name="c", subcore_axis_name="s")

    @functools.partial(
        pl.kernel,
        out_type=jax.ShapeDtypeStruct((ntok * EMB_DIM,), jnp.float32),
        mesh=mesh,
        scratch_types=[
            pltpu.VMEM((NCOLS * EMB_DIM // 2,), jnp.int32),
            pltpu.VMEM((tpw * N_PROJ,), jnp.int32),
            pltpu.VMEM((tpw * EMB_DIM,), jnp.float32),
        ],
        compiler_params=pltpu.CompilerParams(needs_layout_passes=False),
    )
    def bag(table_hbm, addr_hbm, out_hbm, table_v, addr_v, out_v):
        wid = lax.axis_index("s") * nc + lax.axis_index("c")
        pltpu.sync_copy(table_hbm, table_v)
        pltpu.sync_copy(
            addr_hbm.at[pl.ds(wid * (tpw * N_PROJ), tpw * N_PROJ)], addr_v)
        lane = lax.iota(jnp.int32, 16)
        offs = [lane, lane + 16]

        def body(t, carry):
            accs = [jnp.zeros((16,), jnp.float32) for _ in range(EMB_DIM // 16)]
            av = addr_v[pl.ds(t * N_PROJ, N_PROJ)]
            for p in range(N_PROJ):
                a = av[p]
                for g in range(2):
                    w = plsc.load_gather(table_v, [a + offs[g]])
                    pair = plsc.bitcast(w, jnp.bfloat16)
                    lo, hi = plsc.unpack(pair, format=plsc.PackFormat.INTERLEAVED)
                    accs[2 * g] = accs[2 * g] + lo
                    accs[2 * g + 1] = accs[2 * g + 1] + hi
            for c in range(EMB_DIM // 16):
                out_v[pl.ds(t * EMB_DIM + 16 * c, 16)] = accs[c] * (1.0 / N_PROJ)
            return carry

        lax.fori_loop(0, tpw, body, 0)
        pltpu.sync_copy(out_v, out_hbm.at[pl.ds(wid * (tpw * EMB_DIM),
                                                tpw * EMB_DIM)])

    return bag(ptable, addr_flat)


@jax.jit
def kernel(x, projection_mat, emb_weight):
    bs, seq_len, _ = x.shape
    ntok = bs * seq_len
    ctok = ntok // NCHUNK
    xf = x.reshape(ntok, INP_DIM)
    ptable = _pack_table(emb_weight)
    outs = []
    for i in range(NCHUNK):
        addr = _tc_addr(xf, projection_mat, i, NCHUNK)
        outs.append(_sc_bag(ptable, addr.reshape(-1), ctok))
    out = jnp.concatenate(outs)
    return out.reshape(bs, seq_len, EMB_DIM)
